# baseline (speedup 1.0000x reference)
import functools

import jax
import jax.numpy as jnp
from jax import lax
from jax.experimental import pallas as pl
from jax.experimental.pallas import tpu as pltpu
from jax.experimental.pallas import tpu_sc as plsc

_N = 10000
_NP = 10240
_E = 320000
_EB = 2560
_EH1 = 80 * _EB
_EH2 = _E - _EH1
_EXR = _EB // 128
_NB = 2048

_NC = 2
_NS = 16
_NW = _NC * _NS
_RT = _NP // _NS


def _pick_chunk(ew):
    for ch in range(128, 0, -8):
        if ew % ch == 0:
            return ch
    raise ValueError(ew)


def _edge_body(ea, mw0, mb0, mw1, mb1, mw2, mb2,
               aw0, ab0, aw1, ab1, aw2, aw2t, ab2, vals, ex):
    x = ea[...]
    h = jnp.maximum(jnp.dot(x, mw0[...], preferred_element_type=jnp.float32) + mb0[...], 0.0)
    h = jnp.maximum(jnp.dot(h, mw1[...], preferred_element_type=jnp.float32) + mb1[...], 0.0)
    m = jnp.dot(h, mw2[...], preferred_element_type=jnp.float32) + mb2[...]
    a = jnp.maximum(jnp.dot(x, aw0[...], preferred_element_type=jnp.float32) + ab0[...], 0.0)
    a = jnp.maximum(jnp.dot(a, aw1[...], preferred_element_type=jnp.float32) + ab1[...], 0.0)
    lg = jnp.dot(a, aw2[...], preferred_element_type=jnp.float32) + ab2[0, 0]
    e = jnp.exp(lg)
    vals[...] = m * e
    ex[...] = e.reshape(1, _EXR, 128)


def _edge_stage(edge_attr, mw0, mb0, mw1, mb1, mw2, mb2, aw0, ab0, aw1, ab1, aw2, aw2t, ab2):
    ne = edge_attr.shape[0]
    grid = ne // _EB
    full = lambda shape: pl.BlockSpec(shape, lambda i: (0,) * len(shape))
    return pl.pallas_call(
        _edge_body,
        grid=(grid,),
        in_specs=[
            pl.BlockSpec((_EB, 16), lambda i: (i, 0)),
            full((16, 256)), full((1, 256)),
            full((256, 256)), full((1, 256)),
            full((256, 128)), full((1, 128)),
            full((16, 128)), full((1, 128)),
            full((128, 128)), full((1, 128)),
            full((128, 1)), full((1, 128)), full((1, 1)),
        ],
        out_specs=[
            pl.BlockSpec((_EB, 128), lambda i: (i, 0)),
            pl.BlockSpec((1, _EXR, 128), lambda i: (i, 0, 0)),
        ],
        out_shape=[
            jax.ShapeDtypeStruct((ne, 128), jnp.float32),
            jax.ShapeDtypeStruct((ne // _EB, _EXR, 128), jnp.float32),
        ],
    )(edge_attr, mw0, mb0, mw1, mb1, mw2, mb2, aw0, ab0, aw1, ab1, aw2, aw2t, ab2)


def _sc_scatter_body(ew, nit, ch, zf, off, vals_hbm, ex_hbm, dst_hbm, agg_out,
                     den_out, idx_a, idx_b, ex_a, ex_b, rows_a, rows_b, z1,
                     agg_sh, den_sh, sem_a, sem_b, ssem_a, ssem_b):
    c = lax.axis_index("c")
    s = lax.axis_index("s")
    wid = c * _NS + s
    _ZV16 = jnp.zeros((16,), jnp.float32)

    def start(j, rows, idx8, ex8, sem):
        base = wid * ew + j * ch
        pltpu.async_copy(vals_hbm.at[pl.ds(base, ch), :], rows, sem)
        pltpu.async_copy(dst_hbm.at[pl.ds(off + base, ch)], idx8, sem)
        pltpu.async_copy(ex_hbm.at[pl.ds(base, ch)], ex8, sem)

    def wait(j, rows, idx8, ex8, sem):
        base = wid * ew + j * ch
        pltpu.make_async_copy(vals_hbm.at[pl.ds(base, ch), :], rows, sem).wait()
        pltpu.make_async_copy(dst_hbm.at[pl.ds(off + base, ch)], idx8, sem).wait()
        pltpu.make_async_copy(ex_hbm.at[pl.ds(base, ch)], ex8, sem).wait()


    def zero2(r, _):
        def zl(k, _):
            rows_a[r, pl.ds(k * 16, 16)] = _ZV16
            return 0
        return lax.fori_loop(0, 8, zl, 0)
    lax.fori_loop(0, ch, zero2, 0)

    def zl1(k, _):
        z1[pl.ds(k * 16, 16)] = _ZV16
        return 0
    lax.fori_loop(0, _RT // 16, zl1, 0)

    def zfill(b, _):
        pltpu.sync_copy(rows_a.at[pl.ds(0, zf), :],
                        agg_sh.at[pl.ds(s * _RT + b * zf, zf), :])
        return 0
    lax.fori_loop(0, _RT // zf, zfill, 0)
    start(0, rows_a, idx_a, ex_a, sem_a)
    pltpu.sync_copy(z1, den_sh.at[pl.ds(s * _RT, _RT)])

    plsc.subcore_barrier()

    def start_scat(rows, idx8, ex8, ssem):
        pltpu.async_copy(rows, agg_sh.at[idx8], ssem, add=True)
        pltpu.sync_copy(ex8, den_sh.at[idx8], add=True)

    def wait_scat(rows, idx8, ssem):
        pltpu.make_async_copy(rows, agg_sh.at[idx8], ssem).wait()

    def body(t, _):
        ja = 2 * t
        wait(ja, rows_a, idx_a, ex_a, sem_a)

        @pl.when(ja + 1 < nit)
        def _():
            @pl.when(ja > 0)
            def _():
                wait_scat(rows_b, idx_b, ssem_b)
            start(ja + 1, rows_b, idx_b, ex_b, sem_b)
        start_scat(rows_a, idx_a, ex_a, ssem_a)

        @pl.when(ja + 1 < nit)
        def _():
            wait(ja + 1, rows_b, idx_b, ex_b, sem_b)

            @pl.when(ja + 2 < nit)
            def _():
                wait_scat(rows_a, idx_a, ssem_a)
                start(ja + 2, rows_a, idx_a, ex_a, sem_a)
            start_scat(rows_b, idx_b, ex_b, ssem_b)
        return 0
    lax.fori_loop(0, (nit + 1) // 2, body, 0)
    wait_scat(rows_a, idx_a, ssem_a)
    if nit > 1:
        wait_scat(rows_b, idx_b, ssem_b)

    plsc.subcore_barrier()

    hro = pltpu.async_copy(agg_sh.at[pl.ds(s * _RT, _RT), :],
                           agg_out.at[c, pl.ds(s * _RT, _RT), :], sem_a)
    pltpu.sync_copy(den_sh.at[pl.ds(s * _RT, _RT)],
                    den_out.at[c, pl.ds(s * _RT, _RT)])
    hro.wait()


@functools.cache
def _sc_scatter(ne, off):
    ew = ne // _NW
    ch = _pick_chunk(ew)
    nit = ew // ch
    zf = next(z for z in (640, 320, 160, 128, 80, 64, 40, 32, 16, 8) if z <= ch)
    return pl.kernel(
        functools.partial(_sc_scatter_body, ew, nit, ch, zf, off),
        mesh=plsc.VectorSubcoreMesh(core_axis_name="c", subcore_axis_name="s",
                                    num_cores=_NC, num_subcores=_NS),
        out_type=[
            jax.ShapeDtypeStruct((_NC, _NP, 128), jnp.float32),
            jax.ShapeDtypeStruct((_NC, _NP), jnp.float32),
        ],
        scratch_types=[
            pltpu.VMEM((ch,), jnp.int32),
            pltpu.VMEM((ch,), jnp.int32),
            pltpu.VMEM((ch,), jnp.float32),
            pltpu.VMEM((ch,), jnp.float32),
            pltpu.VMEM((ch, 128), jnp.float32),
            pltpu.VMEM((ch, 128), jnp.float32),
            pltpu.VMEM((_RT,), jnp.float32),
            pltpu.VMEM_SHARED((_NP, 128), jnp.float32),
            pltpu.VMEM_SHARED((_NP,), jnp.float32),
            pltpu.SemaphoreType.DMA,
            pltpu.SemaphoreType.DMA,
            pltpu.SemaphoreType.DMA,
            pltpu.SemaphoreType.DMA,
        ],
    )


def _upd_body(aggA, aggB, denA, denB, w0, b0, w1, b1, w2, b2, out):
    agg = (aggA[0] + aggA[1]) + (aggB[0] + aggB[1])
    den = (denA[0] + denA[1]) + (denB[0] + denB[1])
    x = agg * (1.0 / (den + 1e-16))
    u = jnp.maximum(jnp.dot(x, w0[...], preferred_element_type=jnp.float32) + b0[...], 0.0)
    u = jnp.maximum(jnp.dot(u, w1[...], preferred_element_type=jnp.float32) + b1[...], 0.0)
    out[...] = jnp.dot(u, w2[...], preferred_element_type=jnp.float32) + b2[...]


def _update_stage(aggA, aggB, denA, denB, w0, b0, w1, b1, w2, b2):
    grid = _NP // _NB
    full = lambda shape: pl.BlockSpec(shape, lambda i: (0,) * len(shape))
    return pl.pallas_call(
        _upd_body,
        grid=(grid,),
        in_specs=[
            pl.BlockSpec((2, _NB, 128), lambda i: (0, i, 0)),
            pl.BlockSpec((2, _NB, 128), lambda i: (0, i, 0)),
            pl.BlockSpec((2, _NB, 1), lambda i: (0, i, 0)),
            pl.BlockSpec((2, _NB, 1), lambda i: (0, i, 0)),
            full((128, 256)), full((1, 256)),
            full((256, 256)), full((1, 256)),
            full((256, 2)), full((1, 2)),
        ],
        out_specs=pl.BlockSpec((_NB, 2), lambda i: (i, 0)),
        out_shape=jax.ShapeDtypeStruct((_NP, 2), jnp.float32),
    )(aggA, aggB, denA, denB, w0, b0, w1, b1, w2, b2)


def kernel(edge_attr, edge_index,
           msg_w0, msg_b0, msg_w1, msg_b1, msg_w2, msg_b2,
           attn_w0, attn_b0, attn_w1, attn_b1, attn_w2, attn_b2,
           upd_w0, upd_b0, upd_w1, upd_b1, upd_w2, upd_b2):
    ws = (msg_w0, msg_b0.reshape(1, -1), msg_w1, msg_b1.reshape(1, -1),
          msg_w2, msg_b2.reshape(1, -1),
          attn_w0, attn_b0.reshape(1, -1), attn_w1, attn_b1.reshape(1, -1),
          attn_w2, attn_w2.reshape(1, -1), attn_b2.reshape(1, 1))
    dst = edge_index[1]
    v1, e1 = _edge_stage(edge_attr[:_EH1], *ws)
    a1, d1 = _sc_scatter(_EH1, 0)(v1, e1.reshape(_EH1), dst)
    v2, e2 = _edge_stage(edge_attr[_EH1:], *ws)
    a2, d2 = _sc_scatter(_EH2, _EH1)(v2, e2.reshape(_EH2), dst)
    out = _update_stage(
        a1, a2, d1.reshape(_NC, _NP, 1), d2.reshape(_NC, _NP, 1),
        upd_w0, upd_b0.reshape(1, -1), upd_w1, upd_b1.reshape(1, -1),
        upd_w2, upd_b2.reshape(1, -1),
    )
    return out[:_N]


# speedup vs baseline: 1.0235x; 1.0235x over previous
import functools

import jax
import jax.numpy as jnp
from jax import lax
from jax.experimental import pallas as pl
from jax.experimental.pallas import tpu as pltpu
from jax.experimental.pallas import tpu_sc as plsc

_N = 10000
_NP = 10240
_E = 320000
_EB = 2560
_EH1 = 80 * _EB
_EH2 = _E - _EH1
_EXR = _EB // 128
_NB = 2048

_NC = 2
_NS = 16
_NW = _NC * _NS
_RT = _NP // _NS


def _pick_chunk(ew):
    for ch in range(128, 0, -8):
        if ew % ch == 0:
            return ch
    raise ValueError(ew)


def _edge_body(ea, mw0, mb0, mw1, mb1, mw2, mb2,
               aw0, ab0, aw1, ab1, aw2, aw2t, ab2, vals, ex):
    x = ea[...]
    h = jnp.maximum(jnp.dot(x, mw0[...], preferred_element_type=jnp.float32) + mb0[...], 0.0)
    h = jnp.maximum(jnp.dot(h, mw1[...], preferred_element_type=jnp.float32) + mb1[...], 0.0)
    m = jnp.dot(h, mw2[...], preferred_element_type=jnp.float32) + mb2[...]
    a = jnp.maximum(jnp.dot(x, aw0[...], preferred_element_type=jnp.float32) + ab0[...], 0.0)
    a = jnp.maximum(jnp.dot(a, aw1[...], preferred_element_type=jnp.float32) + ab1[...], 0.0)
    lg = jnp.dot(a, aw2[...], preferred_element_type=jnp.float32) + ab2[0, 0]
    e = jnp.exp(lg)
    vals[...] = m * e
    ex[...] = e.reshape(1, _EXR, 128)


def _edge_stage(edge_attr, mw0, mb0, mw1, mb1, mw2, mb2, aw0, ab0, aw1, ab1, aw2, aw2t, ab2):
    ne = edge_attr.shape[0]
    grid = ne // _EB
    full = lambda shape: pl.BlockSpec(shape, lambda i: (0,) * len(shape))
    return pl.pallas_call(
        _edge_body,
        grid=(grid,),
        in_specs=[
            pl.BlockSpec((_EB, 16), lambda i: (i, 0)),
            full((16, 256)), full((1, 256)),
            full((256, 256)), full((1, 256)),
            full((256, 128)), full((1, 128)),
            full((16, 128)), full((1, 128)),
            full((128, 128)), full((1, 128)),
            full((128, 1)), full((1, 128)), full((1, 1)),
        ],
        out_specs=[
            pl.BlockSpec((_EB, 128), lambda i: (i, 0)),
            pl.BlockSpec((1, _EXR, 128), lambda i: (i, 0, 0)),
        ],
        out_shape=[
            jax.ShapeDtypeStruct((ne, 128), jnp.float32),
            jax.ShapeDtypeStruct((ne // _EB, _EXR, 128), jnp.float32),
        ],
    )(edge_attr, mw0, mb0, mw1, mb1, mw2, mb2, aw0, ab0, aw1, ab1, aw2, aw2t, ab2)


def _sc_scatter_body(ew, nit, ch, zf, vals_hbm, ex_hbm, dst_hbm, agg_out,
                     den_out, idx_a, idx_b, ex_a, ex_b, rows_a, rows_b, z1,
                     agg_sh, den_sh, sem_a, sem_b, ssem_a, ssem_b):
    c = lax.axis_index("c")
    s = lax.axis_index("s")
    wid = c * _NS + s
    _ZV16 = jnp.zeros((16,), jnp.float32)

    def start(j, rows, idx8, ex8, sem):
        base = wid * ew + j * ch
        pltpu.async_copy(vals_hbm.at[pl.ds(base, ch), :], rows, sem)
        pltpu.async_copy(dst_hbm.at[pl.ds(base, ch)], idx8, sem)
        pltpu.async_copy(ex_hbm.at[pl.ds(base, ch)], ex8, sem)

    def wait(j, rows, idx8, ex8, sem):
        base = wid * ew + j * ch
        pltpu.make_async_copy(vals_hbm.at[pl.ds(base, ch), :], rows, sem).wait()
        pltpu.make_async_copy(dst_hbm.at[pl.ds(base, ch)], idx8, sem).wait()
        pltpu.make_async_copy(ex_hbm.at[pl.ds(base, ch)], ex8, sem).wait()


    def zero2(r, _):
        def zl(k, _):
            rows_a[r, pl.ds(k * 16, 16)] = _ZV16
            return 0
        return lax.fori_loop(0, 8, zl, 0)
    lax.fori_loop(0, ch, zero2, 0)

    def zl1(k, _):
        z1[pl.ds(k * 16, 16)] = _ZV16
        return 0
    lax.fori_loop(0, _RT // 16, zl1, 0)

    def zfill(b, _):
        pltpu.sync_copy(rows_a.at[pl.ds(0, zf), :],
                        agg_sh.at[pl.ds(s * _RT + b * zf, zf), :])
        return 0
    lax.fori_loop(0, _RT // zf, zfill, 0)
    pltpu.sync_copy(z1, den_sh.at[pl.ds(s * _RT, _RT)])

    plsc.subcore_barrier()

    def start_scat(rows, idx8, ex8, ssem):
        pltpu.async_copy(rows, agg_sh.at[idx8], ssem, add=True)
        pltpu.sync_copy(ex8, den_sh.at[idx8], add=True)

    def wait_scat(rows, idx8, ssem):
        pltpu.make_async_copy(rows, agg_sh.at[idx8], ssem).wait()

    start(0, rows_a, idx_a, ex_a, sem_a)

    def body(t, _):
        ja = 2 * t
        wait(ja, rows_a, idx_a, ex_a, sem_a)

        @pl.when(ja + 1 < nit)
        def _():
            @pl.when(ja > 0)
            def _():
                wait_scat(rows_b, idx_b, ssem_b)
            start(ja + 1, rows_b, idx_b, ex_b, sem_b)
        start_scat(rows_a, idx_a, ex_a, ssem_a)

        @pl.when(ja + 1 < nit)
        def _():
            wait(ja + 1, rows_b, idx_b, ex_b, sem_b)

            @pl.when(ja + 2 < nit)
            def _():
                wait_scat(rows_a, idx_a, ssem_a)
                start(ja + 2, rows_a, idx_a, ex_a, sem_a)
            start_scat(rows_b, idx_b, ex_b, ssem_b)
        return 0
    lax.fori_loop(0, (nit + 1) // 2, body, 0)
    wait_scat(rows_a, idx_a, ssem_a)
    if nit > 1:
        wait_scat(rows_b, idx_b, ssem_b)

    plsc.subcore_barrier()

    pltpu.sync_copy(agg_sh.at[pl.ds(s * _RT, _RT), :],
                    agg_out.at[c, pl.ds(s * _RT, _RT), :])
    pltpu.sync_copy(den_sh.at[pl.ds(s * _RT, _RT)],
                    den_out.at[c, pl.ds(s * _RT, _RT)])


@functools.cache
def _sc_scatter(ne):
    ew = ne // _NW
    ch = _pick_chunk(ew)
    nit = ew // ch
    zf = next(z for z in (640, 320, 160, 128, 80, 64, 40, 32, 16, 8) if z <= ch)
    return pl.kernel(
        functools.partial(_sc_scatter_body, ew, nit, ch, zf),
        mesh=plsc.VectorSubcoreMesh(core_axis_name="c", subcore_axis_name="s",
                                    num_cores=_NC, num_subcores=_NS),
        out_type=[
            jax.ShapeDtypeStruct((_NC, _NP, 128), jnp.float32),
            jax.ShapeDtypeStruct((_NC, _NP), jnp.float32),
        ],
        scratch_types=[
            pltpu.VMEM((ch,), jnp.int32),
            pltpu.VMEM((ch,), jnp.int32),
            pltpu.VMEM((ch,), jnp.float32),
            pltpu.VMEM((ch,), jnp.float32),
            pltpu.VMEM((ch, 128), jnp.float32),
            pltpu.VMEM((ch, 128), jnp.float32),
            pltpu.VMEM((_RT,), jnp.float32),
            pltpu.VMEM_SHARED((_NP, 128), jnp.float32),
            pltpu.VMEM_SHARED((_NP,), jnp.float32),
            pltpu.SemaphoreType.DMA,
            pltpu.SemaphoreType.DMA,
            pltpu.SemaphoreType.DMA,
            pltpu.SemaphoreType.DMA,
        ],
    )


def _upd_body(aggA, aggB, denA, denB, w0, b0, w1, b1, w2, b2, out):
    agg = (aggA[0] + aggA[1]) + (aggB[0] + aggB[1])
    den = (denA[0] + denA[1]) + (denB[0] + denB[1])
    x = agg * (1.0 / (den + 1e-16))
    u = jnp.maximum(jnp.dot(x, w0[...], preferred_element_type=jnp.float32) + b0[...], 0.0)
    u = jnp.maximum(jnp.dot(u, w1[...], preferred_element_type=jnp.float32) + b1[...], 0.0)
    out[...] = jnp.dot(u, w2[...], preferred_element_type=jnp.float32) + b2[...]


def _update_stage(aggA, aggB, denA, denB, w0, b0, w1, b1, w2, b2):
    grid = _NP // _NB
    full = lambda shape: pl.BlockSpec(shape, lambda i: (0,) * len(shape))
    return pl.pallas_call(
        _upd_body,
        grid=(grid,),
        in_specs=[
            pl.BlockSpec((2, _NB, 128), lambda i: (0, i, 0)),
            pl.BlockSpec((2, _NB, 128), lambda i: (0, i, 0)),
            pl.BlockSpec((2, _NB, 1), lambda i: (0, i, 0)),
            pl.BlockSpec((2, _NB, 1), lambda i: (0, i, 0)),
            full((128, 256)), full((1, 256)),
            full((256, 256)), full((1, 256)),
            full((256, 2)), full((1, 2)),
        ],
        out_specs=pl.BlockSpec((_NB, 2), lambda i: (i, 0)),
        out_shape=jax.ShapeDtypeStruct((_NP, 2), jnp.float32),
    )(aggA, aggB, denA, denB, w0, b0, w1, b1, w2, b2)


def kernel(edge_attr, edge_index,
           msg_w0, msg_b0, msg_w1, msg_b1, msg_w2, msg_b2,
           attn_w0, attn_b0, attn_w1, attn_b1, attn_w2, attn_b2,
           upd_w0, upd_b0, upd_w1, upd_b1, upd_w2, upd_b2):
    ws = (msg_w0, msg_b0.reshape(1, -1), msg_w1, msg_b1.reshape(1, -1),
          msg_w2, msg_b2.reshape(1, -1),
          attn_w0, attn_b0.reshape(1, -1), attn_w1, attn_b1.reshape(1, -1),
          attn_w2, attn_w2.reshape(1, -1), attn_b2.reshape(1, 1))
    v1, e1 = _edge_stage(edge_attr[:_EH1], *ws)
    a1, d1 = _sc_scatter(_EH1)(v1, e1.reshape(_EH1), edge_index[1, :_EH1])
    v2, e2 = _edge_stage(edge_attr[_EH1:], *ws)
    a2, d2 = _sc_scatter(_EH2)(v2, e2.reshape(_EH2), edge_index[1, _EH1:])
    out = _update_stage(
        a1, a2, d1.reshape(_NC, _NP, 1), d2.reshape(_NC, _NP, 1),
        upd_w0, upd_b0.reshape(1, -1), upd_w1, upd_b1.reshape(1, -1),
        upd_w2, upd_b2.reshape(1, -1),
    )
    return out[:_N]


# speedup vs baseline: 1.0243x; 1.0008x over previous
import functools

import jax
import jax.numpy as jnp
from jax import lax
from jax.experimental import pallas as pl
from jax.experimental.pallas import tpu as pltpu
from jax.experimental.pallas import tpu_sc as plsc

_N = 10000
_NP = 10240
_E = 320000
_EB = 2560
_EH1 = 80 * _EB
_EH2 = _E - _EH1
_EXR = _EB // 128
_NB = 2048

_NC = 2
_NS = 16
_NW = _NC * _NS
_RT = _NP // _NS


def _pick_chunk(ew):
    for ch in range(128, 0, -8):
        if ew % ch == 0:
            return ch
    raise ValueError(ew)


def _edge_body(ea, mw0, mb0, mw1, mb1, mw2, mb2,
               aw0, ab0, aw1, ab1, aw2, ab2, vals, ex):
    x = ea[...]
    h = jnp.maximum(jnp.dot(x, mw0[...], preferred_element_type=jnp.float32) + mb0[...], 0.0)
    h = jnp.maximum(jnp.dot(h, mw1[...], preferred_element_type=jnp.float32) + mb1[...], 0.0)
    m = jnp.dot(h, mw2[...], preferred_element_type=jnp.float32) + mb2[...]
    a = jnp.maximum(jnp.dot(x, aw0[...], preferred_element_type=jnp.float32) + ab0[...], 0.0)
    a = jnp.maximum(jnp.dot(a, aw1[...], preferred_element_type=jnp.float32) + ab1[...], 0.0)
    lg = jnp.dot(a, aw2[...], preferred_element_type=jnp.float32) + ab2[0, 0]
    e = jnp.exp(lg)
    vals[...] = m * e
    ex[...] = e.reshape(1, _EXR, 128)


def _edge_stage(edge_attr, mw0, mb0, mw1, mb1, mw2, mb2, aw0, ab0, aw1, ab1, aw2, ab2):
    ne = edge_attr.shape[0]
    grid = ne // _EB
    full = lambda shape: pl.BlockSpec(shape, lambda i: (0,) * len(shape))
    return pl.pallas_call(
        _edge_body,
        grid=(grid,),
        in_specs=[
            pl.BlockSpec((_EB, 16), lambda i: (i, 0)),
            full((16, 256)), full((1, 256)),
            full((256, 256)), full((1, 256)),
            full((256, 128)), full((1, 128)),
            full((16, 128)), full((1, 128)),
            full((128, 128)), full((1, 128)),
            full((128, 1)), full((1, 1)),
        ],
        out_specs=[
            pl.BlockSpec((_EB, 128), lambda i: (i, 0)),
            pl.BlockSpec((1, _EXR, 128), lambda i: (i, 0, 0)),
        ],
        out_shape=[
            jax.ShapeDtypeStruct((ne, 128), jnp.float32),
            jax.ShapeDtypeStruct((ne // _EB, _EXR, 128), jnp.float32),
        ],
    )(edge_attr, mw0, mb0, mw1, mb1, mw2, mb2, aw0, ab0, aw1, ab1, aw2, ab2)


def _sc_scatter_body(ew, nit, ch, zf, vals_hbm, ex_hbm, dst_hbm, agg_out,
                     den_out, idx_a, idx_b, ex_a, ex_b, rows_a, rows_b, z1,
                     agg_sh, den_sh, sem_a, sem_b, ssem_a, ssem_b):
    c = lax.axis_index("c")
    s = lax.axis_index("s")
    wid = c * _NS + s
    _ZV16 = jnp.zeros((16,), jnp.float32)

    def start(j, rows, idx8, ex8, sem):
        base = wid * ew + j * ch
        pltpu.async_copy(vals_hbm.at[pl.ds(base, ch), :], rows, sem)
        pltpu.async_copy(dst_hbm.at[pl.ds(base, ch)], idx8, sem)
        pltpu.async_copy(ex_hbm.at[pl.ds(base, ch)], ex8, sem)

    def wait(j, rows, idx8, ex8, sem):
        base = wid * ew + j * ch
        pltpu.make_async_copy(vals_hbm.at[pl.ds(base, ch), :], rows, sem).wait()
        pltpu.make_async_copy(dst_hbm.at[pl.ds(base, ch)], idx8, sem).wait()
        pltpu.make_async_copy(ex_hbm.at[pl.ds(base, ch)], ex8, sem).wait()


    def zero2(r, _):
        def zl(k, _):
            rows_a[r, pl.ds(k * 16, 16)] = _ZV16
            return 0
        return lax.fori_loop(0, 8, zl, 0)
    lax.fori_loop(0, ch, zero2, 0)

    def zl1(k, _):
        z1[pl.ds(k * 16, 16)] = _ZV16
        return 0
    lax.fori_loop(0, _RT // 16, zl1, 0)

    def zfill(b, _):
        pltpu.sync_copy(rows_a.at[pl.ds(0, zf), :],
                        agg_sh.at[pl.ds(s * _RT + b * zf, zf), :])
        return 0
    lax.fori_loop(0, _RT // zf, zfill, 0)
    pltpu.sync_copy(z1, den_sh.at[pl.ds(s * _RT, _RT)])

    plsc.subcore_barrier()

    def start_scat(rows, idx8, ex8, ssem):
        pltpu.async_copy(rows, agg_sh.at[idx8], ssem, add=True)
        pltpu.sync_copy(ex8, den_sh.at[idx8], add=True)

    def wait_scat(rows, idx8, ssem):
        pltpu.make_async_copy(rows, agg_sh.at[idx8], ssem).wait()

    start(0, rows_a, idx_a, ex_a, sem_a)

    def body(t, _):
        ja = 2 * t
        wait(ja, rows_a, idx_a, ex_a, sem_a)

        @pl.when(ja + 1 < nit)
        def _():
            @pl.when(ja > 0)
            def _():
                wait_scat(rows_b, idx_b, ssem_b)
            start(ja + 1, rows_b, idx_b, ex_b, sem_b)
        start_scat(rows_a, idx_a, ex_a, ssem_a)

        @pl.when(ja + 1 < nit)
        def _():
            wait(ja + 1, rows_b, idx_b, ex_b, sem_b)

            @pl.when(ja + 2 < nit)
            def _():
                wait_scat(rows_a, idx_a, ssem_a)
                start(ja + 2, rows_a, idx_a, ex_a, sem_a)
            start_scat(rows_b, idx_b, ex_b, ssem_b)
        return 0
    lax.fori_loop(0, (nit + 1) // 2, body, 0)
    wait_scat(rows_a, idx_a, ssem_a)
    if nit > 1:
        wait_scat(rows_b, idx_b, ssem_b)

    plsc.subcore_barrier()

    pltpu.sync_copy(agg_sh.at[pl.ds(s * _RT, _RT), :],
                    agg_out.at[c, pl.ds(s * _RT, _RT), :])
    pltpu.sync_copy(den_sh.at[pl.ds(s * _RT, _RT)],
                    den_out.at[c, pl.ds(s * _RT, _RT)])


@functools.cache
def _sc_scatter(ne):
    ew = ne // _NW
    ch = _pick_chunk(ew)
    nit = ew // ch
    zf = next(z for z in (640, 320, 160, 128, 80, 64, 40, 32, 16, 8) if z <= ch)
    return pl.kernel(
        functools.partial(_sc_scatter_body, ew, nit, ch, zf),
        mesh=plsc.VectorSubcoreMesh(core_axis_name="c", subcore_axis_name="s",
                                    num_cores=_NC, num_subcores=_NS),
        out_type=[
            jax.ShapeDtypeStruct((_NC, _NP, 128), jnp.float32),
            jax.ShapeDtypeStruct((_NC, _NP), jnp.float32),
        ],
        scratch_types=[
            pltpu.VMEM((ch,), jnp.int32),
            pltpu.VMEM((ch,), jnp.int32),
            pltpu.VMEM((ch,), jnp.float32),
            pltpu.VMEM((ch,), jnp.float32),
            pltpu.VMEM((ch, 128), jnp.float32),
            pltpu.VMEM((ch, 128), jnp.float32),
            pltpu.VMEM((_RT,), jnp.float32),
            pltpu.VMEM_SHARED((_NP, 128), jnp.float32),
            pltpu.VMEM_SHARED((_NP,), jnp.float32),
            pltpu.SemaphoreType.DMA,
            pltpu.SemaphoreType.DMA,
            pltpu.SemaphoreType.DMA,
            pltpu.SemaphoreType.DMA,
        ],
    )


def _upd_body(aggA, aggB, denA, denB, w0, b0, w1, b1, w2, b2, out):
    agg = (aggA[0] + aggA[1]) + (aggB[0] + aggB[1])
    den = (denA[0] + denA[1]) + (denB[0] + denB[1])
    x = agg * (1.0 / (den + 1e-16))
    u = jnp.maximum(jnp.dot(x, w0[...], preferred_element_type=jnp.float32) + b0[...], 0.0)
    u = jnp.maximum(jnp.dot(u, w1[...], preferred_element_type=jnp.float32) + b1[...], 0.0)
    out[...] = jnp.dot(u, w2[...], preferred_element_type=jnp.float32) + b2[...]


def _update_stage(aggA, aggB, denA, denB, w0, b0, w1, b1, w2, b2):
    grid = _NP // _NB
    full = lambda shape: pl.BlockSpec(shape, lambda i: (0,) * len(shape))
    return pl.pallas_call(
        _upd_body,
        grid=(grid,),
        in_specs=[
            pl.BlockSpec((2, _NB, 128), lambda i: (0, i, 0)),
            pl.BlockSpec((2, _NB, 128), lambda i: (0, i, 0)),
            pl.BlockSpec((2, _NB, 1), lambda i: (0, i, 0)),
            pl.BlockSpec((2, _NB, 1), lambda i: (0, i, 0)),
            full((128, 256)), full((1, 256)),
            full((256, 256)), full((1, 256)),
            full((256, 2)), full((1, 2)),
        ],
        out_specs=pl.BlockSpec((_NB, 2), lambda i: (i, 0)),
        out_shape=jax.ShapeDtypeStruct((_NP, 2), jnp.float32),
    )(aggA, aggB, denA, denB, w0, b0, w1, b1, w2, b2)


def kernel(edge_attr, edge_index,
           msg_w0, msg_b0, msg_w1, msg_b1, msg_w2, msg_b2,
           attn_w0, attn_b0, attn_w1, attn_b1, attn_w2, attn_b2,
           upd_w0, upd_b0, upd_w1, upd_b1, upd_w2, upd_b2):
    ws = (msg_w0, msg_b0.reshape(1, -1), msg_w1, msg_b1.reshape(1, -1),
          msg_w2, msg_b2.reshape(1, -1),
          attn_w0, attn_b0.reshape(1, -1), attn_w1, attn_b1.reshape(1, -1),
          attn_w2, attn_b2.reshape(1, 1))
    v1, e1 = _edge_stage(edge_attr[:_EH1], *ws)
    a1, d1 = _sc_scatter(_EH1)(v1, e1.reshape(_EH1), edge_index[1, :_EH1])
    v2, e2 = _edge_stage(edge_attr[_EH1:], *ws)
    a2, d2 = _sc_scatter(_EH2)(v2, e2.reshape(_EH2), edge_index[1, _EH1:])
    out = _update_stage(
        a1, a2, d1.reshape(_NC, _NP, 1), d2.reshape(_NC, _NP, 1),
        upd_w0, upd_b0.reshape(1, -1), upd_w1, upd_b1.reshape(1, -1),
        upd_w2, upd_b2.reshape(1, -1),
    )
    return out[:_N]
